# baseline (speedup 1.0000x reference)
import jax
import jax.numpy as jnp
from jax.experimental import pallas as pl
from jax.experimental.pallas import tpu as pltpu

B = 8192
D_IN = 2048
H = 128
H2 = 64
D_OUT = 1024
E = 8
GH = 64
TOP_K = 2

BT = 1024
KC = 2
KS = D_IN // KC


def _moe_kernel(x_ref, gW1_ref, gb1_ref, gW2_ref, gb2_ref,
                W1_ref, b1_ref, W2_ref, b2_ref, W3_ref, b3_ref, out_ref,
                gh_acc, h1_acc):
    k = pl.program_id(1)
    xt = x_ref[:]
    xb = xt.astype(jnp.bfloat16)

    gh_part = jnp.dot(xt, gW1_ref[:], preferred_element_type=jnp.float32)
    h1_part = jnp.dot(xb, W1_ref[:], preferred_element_type=jnp.float32)

    @pl.when(k == 0)
    def _():
        gh_acc[:] = gh_part
        h1_acc[:] = h1_part

    @pl.when(k != 0)
    def _():
        gh_acc[:] += gh_part
        h1_acc[:] += h1_part

    @pl.when(k == KC - 1)
    def _():
        gh = jnp.maximum(gh_acc[:] + gb1_ref[:][None, :], 0.0)
        logits = jnp.dot(gh, gW2_ref[:], preferred_element_type=jnp.float32) \
            + gb2_ref[:][None, :]
        eids = jax.lax.broadcasted_iota(jnp.int32, (BT, E), 1)
        i1 = jnp.argmax(logits, axis=-1).astype(jnp.int32)
        v1 = jnp.max(logits, axis=-1)
        masked = jnp.where(eids == i1[:, None], -jnp.inf, logits)
        i2 = jnp.argmax(masked, axis=-1).astype(jnp.int32)
        v2 = jnp.max(masked, axis=-1)
        g1 = jax.nn.sigmoid(v1 - v2)
        g2 = 1.0 - g1
        gates = jnp.where(eids == i1[:, None], g1[:, None], 0.0) \
            + jnp.where(eids == i2[:, None], g2[:, None], 0.0)

        h1 = jnp.maximum(h1_acc[:] + b1_ref[:][None, :], 0.0)
        h2s = []
        for e in range(E):
            h2 = jnp.dot(h1[:, e * H:(e + 1) * H].astype(jnp.bfloat16),
                         W2_ref[e], preferred_element_type=jnp.float32)
            h2 = jnp.maximum(h2 + b2_ref[e][None, :], 0.0)
            h2s.append(gates[:, e][:, None] * h2)
        h2cat = jnp.concatenate(h2s, axis=1)
        y = jnp.dot(h2cat.astype(jnp.bfloat16), W3_ref[:],
                    preferred_element_type=jnp.float32)
        y = y + jnp.dot(gates, b3_ref[:], preferred_element_type=jnp.float32)
        out_ref[:] = y


@jax.jit
def kernel(x, gW1, gb1, gW2, gb2, W1, b1, W2, b2, W3, b3):
    W1c = jnp.transpose(W1, (1, 0, 2)).reshape(D_IN, E * H).astype(jnp.bfloat16)
    b1c = b1.reshape(E * H)
    W2b = W2.astype(jnp.bfloat16)
    W3c = W3.reshape(E * H2, D_OUT).astype(jnp.bfloat16)
    grid = (B // BT, KC)
    full = lambda shape: pl.BlockSpec(shape, lambda i, k: (0,) * len(shape))
    return pl.pallas_call(
        _moe_kernel,
        grid=grid,
        in_specs=[
            pl.BlockSpec((BT, KS), lambda i, k: (i, k)),
            pl.BlockSpec((KS, GH), lambda i, k: (k, 0)),
            full((GH,)), full((GH, E)), full((E,)),
            pl.BlockSpec((KS, E * H), lambda i, k: (k, 0)),
            full((E * H,)),
            full((E, H, H2)), full((E, H2)),
            full((E * H2, D_OUT)), full((E, D_OUT)),
        ],
        out_specs=pl.BlockSpec((BT, D_OUT), lambda i, k: (i, 0)),
        out_shape=jax.ShapeDtypeStruct((B, D_OUT), jnp.float32),
        scratch_shapes=[pltpu.VMEM((BT, GH), jnp.float32),
                        pltpu.VMEM((BT, E * H), jnp.float32)],
    )(x, gW1, gb1, gW2, gb2, W1c, b1c, W2b, b2, W3c, b3)


# speedup vs baseline: 1.0561x; 1.0561x over previous
import jax
import jax.numpy as jnp
from jax.experimental import pallas as pl

B = 8192
D_IN = 2048
H = 128
H2 = 64
D_OUT = 1024
E = 8
GH = 64
TOP_K = 2

BT = 1024


def _moe_kernel(x_ref, gW1_ref, gb1_ref, gW2_ref, gb2_ref,
                W1_ref, b1_ref, W2_ref, b2_ref, W3_ref, b3_ref, out_ref):
    xt = x_ref[:]

    gh = jnp.maximum(
        jnp.dot(xt, gW1_ref[:], preferred_element_type=jnp.float32)
        + gb1_ref[:][None, :], 0.0)
    logits = jnp.dot(gh, gW2_ref[:], preferred_element_type=jnp.float32) \
        + gb2_ref[:][None, :]

    eids = jax.lax.broadcasted_iota(jnp.int32, (BT, E), 1)
    i1 = jnp.argmax(logits, axis=-1).astype(jnp.int32)
    v1 = jnp.max(logits, axis=-1)
    masked = jnp.where(eids == i1[:, None], -jnp.inf, logits)
    i2 = jnp.argmax(masked, axis=-1).astype(jnp.int32)
    v2 = jnp.max(masked, axis=-1)
    g1 = jax.nn.sigmoid(v1 - v2)
    g2 = 1.0 - g1
    gates = jnp.where(eids == i1[:, None], g1[:, None], 0.0) \
        + jnp.where(eids == i2[:, None], g2[:, None], 0.0)

    xb = xt.astype(jnp.bfloat16)
    h1 = jnp.dot(xb, W1_ref[:], preferred_element_type=jnp.float32)
    h1 = jnp.maximum(h1 + b1_ref[:][None, :], 0.0)
    h2s = []
    for e in range(E):
        h2 = jnp.dot(h1[:, e * H:(e + 1) * H].astype(jnp.bfloat16),
                     W2_ref[e], preferred_element_type=jnp.float32)
        h2 = jnp.maximum(h2 + b2_ref[e][None, :], 0.0)
        h2s.append(gates[:, e][:, None] * h2)
    h2cat = jnp.concatenate(h2s, axis=1)
    y = jnp.dot(h2cat.astype(jnp.bfloat16), W3_ref[:],
                preferred_element_type=jnp.float32)
    y = y + jnp.dot(gates, b3_ref[:], preferred_element_type=jnp.float32)
    out_ref[:] = y


@jax.jit
def kernel(x, gW1, gb1, gW2, gb2, W1, b1, W2, b2, W3, b3):
    W1c = jnp.transpose(W1, (1, 0, 2)).reshape(D_IN, E * H).astype(jnp.bfloat16)
    b1c = b1.reshape(E * H)
    W2b = W2.astype(jnp.bfloat16)
    W3c = W3.reshape(E * H2, D_OUT).astype(jnp.bfloat16)
    grid = (B // BT,)
    full = lambda shape: pl.BlockSpec(shape, lambda i: (0,) * len(shape))
    return pl.pallas_call(
        _moe_kernel,
        grid=grid,
        in_specs=[
            pl.BlockSpec((BT, D_IN), lambda i: (i, 0)),
            full((D_IN, GH)), full((GH,)), full((GH, E)), full((E,)),
            full((D_IN, E * H)), full((E * H,)),
            full((E, H, H2)), full((E, H2)),
            full((E * H2, D_OUT)), full((E, D_OUT)),
        ],
        out_specs=pl.BlockSpec((BT, D_OUT), lambda i: (i, 0)),
        out_shape=jax.ShapeDtypeStruct((B, D_OUT), jnp.float32),
    )(x, gW1, gb1, gW2, gb2, W1c, b1c, W2b, b2, W3c, b3)
